# Initial kernel scaffold; baseline (speedup 1.0000x reference)
import jax
import jax.numpy as jnp
from jax.experimental import pallas as pl


def kernel(x, batch, params):
    raise NotImplementedError("write your pallas kernel here")


# speedup vs baseline: 2.5445x; 2.5445x over previous
import functools

import jax
import jax.numpy as jnp
from jax import lax
from jax.experimental import pallas as pl

B, P, K = 32, 128, 16
EPS = 1e-5
_HI = lax.Precision.HIGHEST


def _edge_conv(h, wd, wb, b1, w2, b2):
    sq = jnp.sum(h * h, axis=1, keepdims=True)
    g = lax.dot_general(h, h, (((1,), (1,)), ((), ())), precision=_HI)
    d = sq + sq.T - 2.0 * g

    a = jnp.dot(h, wd, precision=_HI) + b1
    bv = jnp.dot(h, wb, precision=_HI)

    col = lax.broadcasted_iota(jnp.int32, (P, P), 1)
    out0 = jnp.zeros((P, w2.shape[1]), jnp.float32)

    def body(_, carry):
        dw, out = carry
        m = jnp.min(dw, axis=1, keepdims=True)
        eq = dw == m
        first = jnp.min(jnp.where(eq, col, P), axis=1, keepdims=True)
        oh = col == first
        dw = jnp.where(oh, jnp.float32(jnp.inf), dw)
        gk = jnp.dot(oh.astype(jnp.float32), bv, precision=_HI)
        r1 = jnp.maximum(a + gk, 0.0)
        h2 = jnp.dot(r1, w2, precision=_HI) + b2
        return dw, jnp.maximum(out, jnp.maximum(h2, 0.0))

    _, out = lax.fori_loop(0, K, body, (d, out0))
    return out


def _pnet_kernel(x_ref,
                 wd0, wb0, b10, w20, b20,
                 wd1, wb1, b11, w21, b21,
                 wd2, wb2, b12, w22, b22,
                 wc1, bc1, wc2, bc2, wc3, bc3,
                 out_ref):
    h = x_ref[0]
    h = _edge_conv(h, wd0[...], wb0[...], b10[...], w20[...], b20[...])
    h = _edge_conv(h, wd1[...], wb1[...], b11[...], w21[...], b21[...])
    h = _edge_conv(h, wd2[...], wb2[...], b12[...], w22[...], b22[...])
    mean = jnp.mean(h, axis=0, keepdims=True)
    mx = jnp.max(h, axis=0, keepdims=True)
    z = jnp.concatenate([mean, mx], axis=1)
    z = jnp.maximum(jnp.dot(z, wc1[...], precision=_HI) + bc1[...], 0.0)
    z = jnp.maximum(jnp.dot(z, wc2[...], precision=_HI) + bc2[...], 0.0)
    out_ref[0] = jnp.dot(z, wc3[...], precision=_HI) + bc3[...]


def _fold_edge(p):
    s1 = p["g1"] / jnp.sqrt(1.0 + EPS)
    w1f = p["W1"] * s1[None, :]
    b1f = p["b1"] * s1 + p["be1"]
    c = p["W1"].shape[0] // 2
    w1a, w1b = w1f[:c], w1f[c:]
    s2 = p["g2"] / jnp.sqrt(1.0 + EPS)
    w2f = p["W2"] * s2[None, :]
    b2f = p["b2"] * s2 + p["be2"]
    return w1a - w1b, w1b, b1f[None, :], w2f, b2f[None, :]


def kernel(x, batch, params):
    xb = jnp.pad(x.reshape(B, P, 6), ((0, 0), (0, 0), (0, 2)))

    wd0, wb0, b10, w20, b20 = _fold_edge(params["conv0"])
    wd0 = jnp.pad(wd0, ((0, 2), (0, 0)))
    wb0 = jnp.pad(wb0, ((0, 2), (0, 0)))
    wd1, wb1, b11, w21, b21 = _fold_edge(params["conv1"])
    wd2, wb2, b12, w22, b22 = _fold_edge(params["conv2"])

    c = params["cls"]
    s1 = c["g1"] / jnp.sqrt(1.0 + EPS)
    wc1 = c["W1"] * s1[None, :]
    bc1 = (c["b1"] * s1 + c["be1"])[None, :]
    s2 = c["g2"] / jnp.sqrt(1.0 + EPS)
    wc2 = c["W2"] * s2[None, :]
    bc2 = (c["b2"] * s2 + c["be2"])[None, :]
    wc3 = jnp.pad(c["W3"], ((0, 0), (0, 126)))
    bc3 = jnp.pad(c["b3"], (0, 126))[None, :]

    ops = [wd0, wb0, b10, w20, b20,
           wd1, wb1, b11, w21, b21,
           wd2, wb2, b12, w22, b22,
           wc1, bc1, wc2, bc2, wc3, bc3]

    def full(a):
        return pl.BlockSpec(a.shape, lambda i: (0,) * a.ndim)

    out = pl.pallas_call(
        _pnet_kernel,
        grid=(B,),
        in_specs=[pl.BlockSpec((1, P, 8), lambda i: (i, 0, 0))] +
                 [full(a) for a in ops],
        out_specs=pl.BlockSpec((1, 1, 128), lambda i: (i, 0, 0)),
        out_shape=jax.ShapeDtypeStruct((B, 1, 128), jnp.float32),
    )(xb, *ops)
    return out.reshape(B, 128)[:, :2]


# speedup vs baseline: 3.5938x; 1.4124x over previous
import functools

import jax
import jax.numpy as jnp
from jax import lax
from jax.experimental import pallas as pl
from jax.experimental.pallas import tpu as pltpu

B, P, K = 32, 128, 16
EPS = 1e-5
_HI = lax.Precision.HIGHEST


def _edge_conv(h, wd, wb, b1, w2, b2, ohs_ref):
    cout = w2.shape[1]
    sq = jnp.sum(h * h, axis=1, keepdims=True)
    g = lax.dot_general(h, h, (((1,), (1,)), ((), ())), precision=_HI)
    d = sq + sq.T - 2.0 * g

    a = jnp.dot(h, wd, precision=_HI) + b1
    bv = jnp.dot(h, wb, precision=_HI)

    col = lax.broadcasted_iota(jnp.int32, (P, P), 1)

    def body(k, dw):
        m = jnp.min(dw, axis=1, keepdims=True)
        eq = dw == m
        first = jnp.min(jnp.where(eq, col, P), axis=1, keepdims=True)
        oh = col == first
        ohs_ref[k] = oh.astype(jnp.float32)
        return jnp.where(oh, jnp.float32(jnp.inf), dw)

    lax.fori_loop(0, K, body, d)

    ohs = ohs_ref[...]
    gk = jnp.dot(ohs.reshape(K * P, P), bv, precision=_HI)
    a_rep = jnp.broadcast_to(a[None], (K, P, cout)).reshape(K * P, cout)
    r1 = jnp.maximum(a_rep + gk, 0.0)
    h2 = jnp.dot(r1, w2, precision=_HI) + b2
    return jnp.max(jnp.maximum(h2, 0.0).reshape(K, P, cout), axis=0)


def _pnet_kernel(x_ref,
                 wd0, wb0, b10, w20, b20,
                 wd1, wb1, b11, w21, b21,
                 wd2, wb2, b12, w22, b22,
                 wc1, bc1, wc2, bc2, wc3, bc3,
                 out_ref, ohs_ref):
    h = x_ref[0]
    h = _edge_conv(h, wd0[...], wb0[...], b10[...], w20[...], b20[...], ohs_ref)
    h = _edge_conv(h, wd1[...], wb1[...], b11[...], w21[...], b21[...], ohs_ref)
    h = _edge_conv(h, wd2[...], wb2[...], b12[...], w22[...], b22[...], ohs_ref)
    mean = jnp.mean(h, axis=0, keepdims=True)
    mx = jnp.max(h, axis=0, keepdims=True)
    z = jnp.concatenate([mean, mx], axis=1)
    z = jnp.maximum(jnp.dot(z, wc1[...], precision=_HI) + bc1[...], 0.0)
    z = jnp.maximum(jnp.dot(z, wc2[...], precision=_HI) + bc2[...], 0.0)
    out_ref[0] = jnp.dot(z, wc3[...], precision=_HI) + bc3[...]


def _fold_edge(p):
    s1 = p["g1"] / jnp.sqrt(1.0 + EPS)
    w1f = p["W1"] * s1[None, :]
    b1f = p["b1"] * s1 + p["be1"]
    c = p["W1"].shape[0] // 2
    w1a, w1b = w1f[:c], w1f[c:]
    s2 = p["g2"] / jnp.sqrt(1.0 + EPS)
    w2f = p["W2"] * s2[None, :]
    b2f = p["b2"] * s2 + p["be2"]
    return w1a - w1b, w1b, b1f[None, :], w2f, b2f[None, :]


def kernel(x, batch, params):
    xb = jnp.pad(x.reshape(B, P, 6), ((0, 0), (0, 0), (0, 2)))

    wd0, wb0, b10, w20, b20 = _fold_edge(params["conv0"])
    wd0 = jnp.pad(wd0, ((0, 2), (0, 0)))
    wb0 = jnp.pad(wb0, ((0, 2), (0, 0)))
    wd1, wb1, b11, w21, b21 = _fold_edge(params["conv1"])
    wd2, wb2, b12, w22, b22 = _fold_edge(params["conv2"])

    c = params["cls"]
    s1 = c["g1"] / jnp.sqrt(1.0 + EPS)
    wc1 = c["W1"] * s1[None, :]
    bc1 = (c["b1"] * s1 + c["be1"])[None, :]
    s2 = c["g2"] / jnp.sqrt(1.0 + EPS)
    wc2 = c["W2"] * s2[None, :]
    bc2 = (c["b2"] * s2 + c["be2"])[None, :]
    wc3 = jnp.pad(c["W3"], ((0, 0), (0, 126)))
    bc3 = jnp.pad(c["b3"], (0, 126))[None, :]

    ops = [wd0, wb0, b10, w20, b20,
           wd1, wb1, b11, w21, b21,
           wd2, wb2, b12, w22, b22,
           wc1, bc1, wc2, bc2, wc3, bc3]

    def full(a):
        return pl.BlockSpec(a.shape, lambda i: (0,) * a.ndim)

    out = pl.pallas_call(
        _pnet_kernel,
        grid=(B,),
        in_specs=[pl.BlockSpec((1, P, 8), lambda i: (i, 0, 0))] +
                 [full(a) for a in ops],
        out_specs=pl.BlockSpec((1, 1, 128), lambda i: (i, 0, 0)),
        out_shape=jax.ShapeDtypeStruct((B, 1, 128), jnp.float32),
        scratch_shapes=[pltpu.VMEM((K, P, P), jnp.float32)],
    )(xb, *ops)
    return out.reshape(B, 128)[:, :2]


# speedup vs baseline: 5.3135x; 1.4785x over previous
import jax
import jax.numpy as jnp
from jax import lax
from jax.experimental import pallas as pl
from jax.experimental.pallas import tpu as pltpu

B, P, K = 32, 128, 16
G = 4
EPS = 1e-5
_HI = lax.Precision.HIGHEST


def _edge_conv(h, wd, wb, b1, w2, b2, ohs_ref):
    cin = h.shape[-1]
    cout = w2.shape[1]
    sq = jnp.sum(h * h, axis=2, keepdims=True)
    hf = h.reshape(G * P, cin)
    a = (jnp.dot(hf, wd, precision=_HI) + b1).reshape(G, P, cout)
    bv = jnp.dot(hf, wb, precision=_HI).reshape(G, P, cout)

    d = jnp.concatenate(
        [(sq[g] + sq[g].T
          - 2.0 * lax.dot_general(h[g], h[g], (((1,), (1,)), ((), ())),
                                  precision=_HI))[None]
         for g in range(G)], axis=0)

    col = lax.broadcasted_iota(jnp.int32, (G, P, P), 2)

    def body(k, dw):
        m = jnp.min(dw, axis=2, keepdims=True)
        eq = dw == m
        first = jnp.min(jnp.where(eq, col, P), axis=2, keepdims=True)
        oh = col == first
        ohs_ref[k] = oh.astype(jnp.float32)
        return jnp.where(oh, jnp.float32(jnp.inf), dw)

    lax.fori_loop(0, K, body, d, unroll=2)

    ohs = ohs_ref[...]
    gk = jnp.concatenate(
        [jnp.dot(ohs[:, g].reshape(K * P, P), bv[g], precision=_HI)
         for g in range(G)], axis=0)
    a_rep = jnp.broadcast_to(a[:, None], (G, K, P, cout)).reshape(
        G * K * P, cout)
    r1 = jnp.maximum(a_rep + gk, 0.0)
    h2 = jnp.dot(r1, w2, precision=_HI) + b2
    return jnp.max(jnp.maximum(h2, 0.0).reshape(G, K, P, cout), axis=1)


def _pnet_kernel(x_ref,
                 wd0, wb0, b10, w20, b20,
                 wd1, wb1, b11, w21, b21,
                 wd2, wb2, b12, w22, b22,
                 wc1, bc1, wc2, bc2, wc3, bc3,
                 out_ref, ohs_ref):
    h = x_ref[...]
    h = _edge_conv(h, wd0[...], wb0[...], b10[...], w20[...], b20[...], ohs_ref)
    h = _edge_conv(h, wd1[...], wb1[...], b11[...], w21[...], b21[...], ohs_ref)
    h = _edge_conv(h, wd2[...], wb2[...], b12[...], w22[...], b22[...], ohs_ref)
    mean = jnp.mean(h, axis=1)
    mx = jnp.max(h, axis=1)
    z = jnp.concatenate([mean, mx], axis=1)
    z = jnp.maximum(jnp.dot(z, wc1[...], precision=_HI) + bc1[...], 0.0)
    z = jnp.maximum(jnp.dot(z, wc2[...], precision=_HI) + bc2[...], 0.0)
    out_ref[...] = (jnp.dot(z, wc3[...], precision=_HI) + bc3[...])[:, None]


def _fold_edge(p):
    s1 = p["g1"] / jnp.sqrt(1.0 + EPS)
    w1f = p["W1"] * s1[None, :]
    b1f = p["b1"] * s1 + p["be1"]
    c = p["W1"].shape[0] // 2
    w1a, w1b = w1f[:c], w1f[c:]
    s2 = p["g2"] / jnp.sqrt(1.0 + EPS)
    w2f = p["W2"] * s2[None, :]
    b2f = p["b2"] * s2 + p["be2"]
    return w1a - w1b, w1b, b1f[None, :], w2f, b2f[None, :]


def kernel(x, batch, params):
    xb = jnp.pad(x.reshape(B, P, 6), ((0, 0), (0, 0), (0, 2)))

    wd0, wb0, b10, w20, b20 = _fold_edge(params["conv0"])
    wd0 = jnp.pad(wd0, ((0, 2), (0, 0)))
    wb0 = jnp.pad(wb0, ((0, 2), (0, 0)))
    wd1, wb1, b11, w21, b21 = _fold_edge(params["conv1"])
    wd2, wb2, b12, w22, b22 = _fold_edge(params["conv2"])

    c = params["cls"]
    s1 = c["g1"] / jnp.sqrt(1.0 + EPS)
    wc1 = c["W1"] * s1[None, :]
    bc1 = (c["b1"] * s1 + c["be1"])[None, :]
    s2 = c["g2"] / jnp.sqrt(1.0 + EPS)
    wc2 = c["W2"] * s2[None, :]
    bc2 = (c["b2"] * s2 + c["be2"])[None, :]
    wc3 = jnp.pad(c["W3"], ((0, 0), (0, 126)))
    bc3 = jnp.pad(c["b3"], (0, 126))[None, :]

    ops = [wd0, wb0, b10, w20, b20,
           wd1, wb1, b11, w21, b21,
           wd2, wb2, b12, w22, b22,
           wc1, bc1, wc2, bc2, wc3, bc3]

    def full(a):
        return pl.BlockSpec(a.shape, lambda i: (0,) * a.ndim)

    out = pl.pallas_call(
        _pnet_kernel,
        grid=(B // G,),
        in_specs=[pl.BlockSpec((G, P, 8), lambda i: (i, 0, 0))] +
                 [full(a) for a in ops],
        out_specs=pl.BlockSpec((G, 1, 128), lambda i: (i, 0, 0)),
        out_shape=jax.ShapeDtypeStruct((B, 1, 128), jnp.float32),
        scratch_shapes=[pltpu.VMEM((K, G, P, P), jnp.float32)],
    )(xb, *ops)
    return out.reshape(B, 128)[:, :2]


# speedup vs baseline: 13.9148x; 2.6188x over previous
import jax
import jax.numpy as jnp
from jax import lax
from jax.experimental import pallas as pl
from jax.experimental.pallas import tpu as pltpu

B, P, K = 32, 128, 16
G = 4
EPS = 1e-5
_HI = lax.Precision.HIGHEST
_MED = lax.Precision.DEFAULT


def _edge_conv(h, wd, wb, b1, w2, b2, ohs_ref):
    cin = h.shape[-1]
    cout = w2.shape[1]
    sq = jnp.sum(h * h, axis=2, keepdims=True)
    hf = h.reshape(G * P, cin)
    a = (jnp.dot(hf, wd, precision=_HI) + b1).reshape(G, P, cout)
    bv = jnp.dot(hf, wb, precision=_HI).reshape(G, P, cout)

    d = jnp.concatenate(
        [(sq[g] + sq[g].T
          - 2.0 * lax.dot_general(h[g], h[g], (((1,), (1,)), ((), ())),
                                  precision=_HI))[None]
         for g in range(G)], axis=0)

    col = lax.broadcasted_iota(jnp.int32, (G, P, P), 2)

    def body(k, dw):
        m = jnp.min(dw, axis=2, keepdims=True)
        eq = dw == m
        first = jnp.min(jnp.where(eq, col, P), axis=2, keepdims=True)
        oh = col == first
        ohs_ref[k] = oh.astype(jnp.float32)
        return jnp.where(oh, jnp.float32(jnp.inf), dw)

    lax.fori_loop(0, K, body, d, unroll=4)

    ohs = ohs_ref[...]
    gk = jnp.concatenate(
        [jnp.dot(ohs[:, g].reshape(K * P, P), bv[g], precision=_MED)
         for g in range(G)], axis=0)
    a_rep = jnp.broadcast_to(a[:, None], (G, K, P, cout)).reshape(
        G * K * P, cout)
    r1 = jnp.maximum(a_rep + gk, 0.0)
    h2 = jnp.dot(r1, w2, precision=_MED) + b2
    return jnp.max(jnp.maximum(h2, 0.0).reshape(G, K, P, cout), axis=1)


def _pnet_kernel(x_ref,
                 wd0, wb0, b10, w20, b20,
                 wd1, wb1, b11, w21, b21,
                 wd2, wb2, b12, w22, b22,
                 wc1, bc1, wc2, bc2, wc3, bc3,
                 out_ref, ohs_ref):
    h = x_ref[...]
    h = _edge_conv(h, wd0[...], wb0[...], b10[...], w20[...], b20[...], ohs_ref)
    h = _edge_conv(h, wd1[...], wb1[...], b11[...], w21[...], b21[...], ohs_ref)
    h = _edge_conv(h, wd2[...], wb2[...], b12[...], w22[...], b22[...], ohs_ref)
    mean = jnp.mean(h, axis=1)
    mx = jnp.max(h, axis=1)
    z = jnp.concatenate([mean, mx], axis=1)
    z = jnp.maximum(jnp.dot(z, wc1[...], precision=_HI) + bc1[...], 0.0)
    z = jnp.maximum(jnp.dot(z, wc2[...], precision=_HI) + bc2[...], 0.0)
    out_ref[...] = (jnp.dot(z, wc3[...], precision=_HI) + bc3[...])[:, None]


def _fold_edge(p):
    s1 = p["g1"] / jnp.sqrt(1.0 + EPS)
    w1f = p["W1"] * s1[None, :]
    b1f = p["b1"] * s1 + p["be1"]
    c = p["W1"].shape[0] // 2
    w1a, w1b = w1f[:c], w1f[c:]
    s2 = p["g2"] / jnp.sqrt(1.0 + EPS)
    w2f = p["W2"] * s2[None, :]
    b2f = p["b2"] * s2 + p["be2"]
    return w1a - w1b, w1b, b1f[None, :], w2f, b2f[None, :]


def kernel(x, batch, params):
    xb = jnp.pad(x.reshape(B, P, 6), ((0, 0), (0, 0), (0, 2)))

    wd0, wb0, b10, w20, b20 = _fold_edge(params["conv0"])
    wd0 = jnp.pad(wd0, ((0, 2), (0, 0)))
    wb0 = jnp.pad(wb0, ((0, 2), (0, 0)))
    wd1, wb1, b11, w21, b21 = _fold_edge(params["conv1"])
    wd2, wb2, b12, w22, b22 = _fold_edge(params["conv2"])

    c = params["cls"]
    s1 = c["g1"] / jnp.sqrt(1.0 + EPS)
    wc1 = c["W1"] * s1[None, :]
    bc1 = (c["b1"] * s1 + c["be1"])[None, :]
    s2 = c["g2"] / jnp.sqrt(1.0 + EPS)
    wc2 = c["W2"] * s2[None, :]
    bc2 = (c["b2"] * s2 + c["be2"])[None, :]
    wc3 = jnp.pad(c["W3"], ((0, 0), (0, 126)))
    bc3 = jnp.pad(c["b3"], (0, 126))[None, :]

    ops = [wd0, wb0, b10, w20, b20,
           wd1, wb1, b11, w21, b21,
           wd2, wb2, b12, w22, b22,
           wc1, bc1, wc2, bc2, wc3, bc3]

    def full(a):
        return pl.BlockSpec(a.shape, lambda i: (0,) * a.ndim)

    out = pl.pallas_call(
        _pnet_kernel,
        grid=(B // G,),
        in_specs=[pl.BlockSpec((G, P, 8), lambda i: (i, 0, 0))] +
                 [full(a) for a in ops],
        out_specs=pl.BlockSpec((G, 1, 128), lambda i: (i, 0, 0)),
        out_shape=jax.ShapeDtypeStruct((B, 1, 128), jnp.float32),
        scratch_shapes=[pltpu.VMEM((K, G, P, P), jnp.float32)],
    )(xb, *ops)
    return out.reshape(B, 128)[:, :2]


# speedup vs baseline: 15.5192x; 1.1153x over previous
import jax
import jax.numpy as jnp
from jax import lax
from jax.experimental import pallas as pl
from jax.experimental.pallas import tpu as pltpu

B, P, K = 32, 128, 16
G = 4
EPS = 1e-5
_HI = lax.Precision.HIGHEST
_MED = lax.Precision.DEFAULT


def _edge_conv(h, wd, wb, b1, w2, b2, ohs_ref):
    cin = h.shape[-1]
    cout = w2.shape[1]
    sq = jnp.sum(h * h, axis=2, keepdims=True)
    hf = h.reshape(G * P, cin)
    a = (jnp.dot(hf, wd, precision=_HI) + b1).reshape(G, P, cout)
    bv = jnp.dot(hf, wb, precision=_HI).reshape(G, P, cout)

    d = jnp.concatenate(
        [(sq[g] + sq[g].T
          - 2.0 * lax.dot_general(h[g], h[g], (((1,), (1,)), ((), ())),
                                  precision=_HI))[None]
         for g in range(G)], axis=0)

    col = lax.broadcasted_iota(jnp.int32, (G, P, P), 2)

    def body(k, dw):
        m = jnp.min(dw, axis=2, keepdims=True)
        eq = dw == m
        first = jnp.min(jnp.where(eq, col, P), axis=2, keepdims=True)
        oh = col == first
        ohs_ref[k] = oh.astype(jnp.float32)
        return jnp.where(oh, jnp.float32(jnp.inf), dw)

    lax.fori_loop(0, K, body, d, unroll=K)

    ohs = ohs_ref[...]
    gk = jnp.concatenate(
        [jnp.dot(ohs[:, g].reshape(K * P, P), bv[g], precision=_MED)
         for g in range(G)], axis=0)
    a_rep = jnp.broadcast_to(a[:, None], (G, K, P, cout)).reshape(
        G * K * P, cout)
    r1 = jnp.maximum(a_rep + gk, 0.0)
    h2 = jnp.dot(r1, w2, precision=_MED) + b2
    return jnp.max(jnp.maximum(h2, 0.0).reshape(G, K, P, cout), axis=1)


def _pnet_kernel(x_ref,
                 wd0, wb0, b10, w20, b20,
                 wd1, wb1, b11, w21, b21,
                 wd2, wb2, b12, w22, b22,
                 wc1, bc1, wc2, bc2, wc3, bc3,
                 out_ref, ohs_ref):
    h = x_ref[...]
    h = _edge_conv(h, wd0[...], wb0[...], b10[...], w20[...], b20[...], ohs_ref)
    h = _edge_conv(h, wd1[...], wb1[...], b11[...], w21[...], b21[...], ohs_ref)
    h = _edge_conv(h, wd2[...], wb2[...], b12[...], w22[...], b22[...], ohs_ref)
    mean = jnp.mean(h, axis=1)
    mx = jnp.max(h, axis=1)
    z = jnp.concatenate([mean, mx], axis=1)
    z = jnp.maximum(jnp.dot(z, wc1[...], precision=_HI) + bc1[...], 0.0)
    z = jnp.maximum(jnp.dot(z, wc2[...], precision=_HI) + bc2[...], 0.0)
    out_ref[...] = (jnp.dot(z, wc3[...], precision=_HI) + bc3[...])[:, None]


def _fold_edge(p):
    s1 = p["g1"] / jnp.sqrt(1.0 + EPS)
    w1f = p["W1"] * s1[None, :]
    b1f = p["b1"] * s1 + p["be1"]
    c = p["W1"].shape[0] // 2
    w1a, w1b = w1f[:c], w1f[c:]
    s2 = p["g2"] / jnp.sqrt(1.0 + EPS)
    w2f = p["W2"] * s2[None, :]
    b2f = p["b2"] * s2 + p["be2"]
    return w1a - w1b, w1b, b1f[None, :], w2f, b2f[None, :]


def kernel(x, batch, params):
    xb = jnp.pad(x.reshape(B, P, 6), ((0, 0), (0, 0), (0, 2)))

    wd0, wb0, b10, w20, b20 = _fold_edge(params["conv0"])
    wd0 = jnp.pad(wd0, ((0, 2), (0, 0)))
    wb0 = jnp.pad(wb0, ((0, 2), (0, 0)))
    wd1, wb1, b11, w21, b21 = _fold_edge(params["conv1"])
    wd2, wb2, b12, w22, b22 = _fold_edge(params["conv2"])

    c = params["cls"]
    s1 = c["g1"] / jnp.sqrt(1.0 + EPS)
    wc1 = c["W1"] * s1[None, :]
    bc1 = (c["b1"] * s1 + c["be1"])[None, :]
    s2 = c["g2"] / jnp.sqrt(1.0 + EPS)
    wc2 = c["W2"] * s2[None, :]
    bc2 = (c["b2"] * s2 + c["be2"])[None, :]
    wc3 = jnp.pad(c["W3"], ((0, 0), (0, 126)))
    bc3 = jnp.pad(c["b3"], (0, 126))[None, :]

    ops = [wd0, wb0, b10, w20, b20,
           wd1, wb1, b11, w21, b21,
           wd2, wb2, b12, w22, b22,
           wc1, bc1, wc2, bc2, wc3, bc3]

    def full(a):
        return pl.BlockSpec(a.shape, lambda i: (0,) * a.ndim)

    out = pl.pallas_call(
        _pnet_kernel,
        grid=(B // G,),
        in_specs=[pl.BlockSpec((G, P, 8), lambda i: (i, 0, 0))] +
                 [full(a) for a in ops],
        out_specs=pl.BlockSpec((G, 1, 128), lambda i: (i, 0, 0)),
        out_shape=jax.ShapeDtypeStruct((B, 1, 128), jnp.float32),
        scratch_shapes=[pltpu.VMEM((K, G, P, P), jnp.float32)],
    )(xb, *ops)
    return out.reshape(B, 128)[:, :2]


# speedup vs baseline: 18.1190x; 1.1675x over previous
import jax
import jax.numpy as jnp
from jax import lax
from jax.experimental import pallas as pl
from jax.experimental.pallas import tpu as pltpu

B, P, K = 32, 128, 16
G = 4
EPS = 1e-5
_HI = lax.Precision.HIGHEST
_MED = lax.Precision.DEFAULT


def _edge_conv(h, wd, wb, b1, w2, b2, ohs_ref):
    cin = h.shape[-1]
    cout = w2.shape[1]
    sq = jnp.sum(h * h, axis=2, keepdims=True)
    hf = h.reshape(G * P, cin)
    a = (jnp.dot(hf, wd, precision=_HI) + b1).reshape(G, P, cout)
    bv = jnp.dot(hf, wb, precision=_HI).reshape(G, P, cout)

    d = jnp.concatenate(
        [(sq[g] + sq[g].T
          - 2.0 * lax.dot_general(h[g], h[g], (((1,), (1,)), ((), ())),
                                  precision=_HI))[None]
         for g in range(G)], axis=0)

    col = lax.broadcasted_iota(jnp.int32, (G, P, P), 2).astype(jnp.float32)

    def body(k, dw):
        m = jnp.min(dw, axis=2, keepdims=True)
        eq = dw == m
        first = jnp.min(jnp.where(eq, col, jnp.float32(P)), axis=2,
                        keepdims=True)
        oh = col == first
        ohs_ref[k] = oh.astype(jnp.float32)
        return jnp.where(oh, jnp.float32(jnp.inf), dw)

    lax.fori_loop(0, K, body, d, unroll=K)

    ohs = ohs_ref[...]
    gk = jnp.concatenate(
        [jnp.dot(ohs[:, g].reshape(K * P, P), bv[g], precision=_MED)
         for g in range(G)], axis=0)
    a_rep = jnp.broadcast_to(a[:, None], (G, K, P, cout)).reshape(
        G * K * P, cout)
    r1 = jnp.maximum(a_rep + gk, 0.0)
    h2 = jnp.dot(r1, w2, precision=_MED) + b2
    return jnp.max(jnp.maximum(h2, 0.0).reshape(G, K, P, cout), axis=1)


def _pnet_kernel(x_ref,
                 wd0, wb0, b10, w20, b20,
                 wd1, wb1, b11, w21, b21,
                 wd2, wb2, b12, w22, b22,
                 wc1, bc1, wc2, bc2, wc3, bc3,
                 out_ref, ohs_ref):
    h = x_ref[...]
    h = _edge_conv(h, wd0[...], wb0[...], b10[...], w20[...], b20[...], ohs_ref)
    h = _edge_conv(h, wd1[...], wb1[...], b11[...], w21[...], b21[...], ohs_ref)
    h = _edge_conv(h, wd2[...], wb2[...], b12[...], w22[...], b22[...], ohs_ref)
    mean = jnp.mean(h, axis=1)
    mx = jnp.max(h, axis=1)
    z = jnp.concatenate([mean, mx], axis=1)
    z = jnp.maximum(jnp.dot(z, wc1[...], precision=_HI) + bc1[...], 0.0)
    z = jnp.maximum(jnp.dot(z, wc2[...], precision=_HI) + bc2[...], 0.0)
    out_ref[...] = (jnp.dot(z, wc3[...], precision=_HI) + bc3[...])[:, None]


def _fold_edge(p):
    s1 = p["g1"] / jnp.sqrt(1.0 + EPS)
    w1f = p["W1"] * s1[None, :]
    b1f = p["b1"] * s1 + p["be1"]
    c = p["W1"].shape[0] // 2
    w1a, w1b = w1f[:c], w1f[c:]
    s2 = p["g2"] / jnp.sqrt(1.0 + EPS)
    w2f = p["W2"] * s2[None, :]
    b2f = p["b2"] * s2 + p["be2"]
    return w1a - w1b, w1b, b1f[None, :], w2f, b2f[None, :]


def kernel(x, batch, params):
    xb = jnp.pad(x.reshape(B, P, 6), ((0, 0), (0, 0), (0, 2)))

    wd0, wb0, b10, w20, b20 = _fold_edge(params["conv0"])
    wd0 = jnp.pad(wd0, ((0, 2), (0, 0)))
    wb0 = jnp.pad(wb0, ((0, 2), (0, 0)))
    wd1, wb1, b11, w21, b21 = _fold_edge(params["conv1"])
    wd2, wb2, b12, w22, b22 = _fold_edge(params["conv2"])

    c = params["cls"]
    s1 = c["g1"] / jnp.sqrt(1.0 + EPS)
    wc1 = c["W1"] * s1[None, :]
    bc1 = (c["b1"] * s1 + c["be1"])[None, :]
    s2 = c["g2"] / jnp.sqrt(1.0 + EPS)
    wc2 = c["W2"] * s2[None, :]
    bc2 = (c["b2"] * s2 + c["be2"])[None, :]
    wc3 = jnp.pad(c["W3"], ((0, 0), (0, 126)))
    bc3 = jnp.pad(c["b3"], (0, 126))[None, :]

    ops = [wd0, wb0, b10, w20, b20,
           wd1, wb1, b11, w21, b21,
           wd2, wb2, b12, w22, b22,
           wc1, bc1, wc2, bc2, wc3, bc3]

    def full(a):
        return pl.BlockSpec(a.shape, lambda i: (0,) * a.ndim)

    out = pl.pallas_call(
        _pnet_kernel,
        grid=(B // G,),
        in_specs=[pl.BlockSpec((G, P, 8), lambda i: (i, 0, 0))] +
                 [full(a) for a in ops],
        out_specs=pl.BlockSpec((G, 1, 128), lambda i: (i, 0, 0)),
        out_shape=jax.ShapeDtypeStruct((B, 1, 128), jnp.float32),
        scratch_shapes=[pltpu.VMEM((K, G, P, P), jnp.float32)],
    )(xb, *ops)
    return out.reshape(B, 128)[:, :2]


# speedup vs baseline: 18.1756x; 1.0031x over previous
import jax
import jax.numpy as jnp
from jax import lax
from jax.experimental import pallas as pl
from jax.experimental.pallas import tpu as pltpu

B, P, K = 32, 128, 16
G = 4
EPS = 1e-5
_HI = lax.Precision.HIGHEST
_MED = lax.Precision.DEFAULT


def _edge_conv(h, wd, wb, b1, w2, b2, ohs_ref):
    cin = h.shape[-1]
    cout = w2.shape[1]
    sq = jnp.sum(h * h, axis=2, keepdims=True)
    hf = h.reshape(G * P, cin)
    a = (jnp.dot(hf, wd, precision=_HI) + b1).reshape(G, P, cout)
    bv = jnp.dot(hf, wb, precision=_HI).reshape(G, P, cout)

    d = jnp.concatenate(
        [(sq[g] + sq[g].T
          - 2.0 * lax.dot_general(h[g], h[g], (((1,), (1,)), ((), ())),
                                  precision=_HI))[None]
         for g in range(G)], axis=0)

    col = lax.broadcasted_iota(jnp.int32, (P, P), 1).astype(jnp.float32)

    gks = []
    for g in range(G):
        dw = d[g]
        for k in range(K):
            m = jnp.min(dw, axis=1, keepdims=True)
            eq = dw == m
            first = jnp.min(jnp.where(eq, col, jnp.float32(P)), axis=1,
                            keepdims=True)
            oh = col == first
            ohs_ref[g, k] = oh.astype(jnp.float32)
            dw = jnp.where(oh, jnp.float32(jnp.inf), dw)
        gks.append(jnp.dot(ohs_ref[g].reshape(K * P, P), bv[g],
                           precision=_MED))
    gk = jnp.concatenate(gks, axis=0)
    a_rep = jnp.broadcast_to(a[:, None], (G, K, P, cout)).reshape(
        G * K * P, cout)
    r1 = jnp.maximum(a_rep + gk, 0.0)
    h2 = jnp.dot(r1, w2, precision=_MED) + b2
    return jnp.max(jnp.maximum(h2, 0.0).reshape(G, K, P, cout), axis=1)


def _pnet_kernel(x_ref,
                 wd0, wb0, b10, w20, b20,
                 wd1, wb1, b11, w21, b21,
                 wd2, wb2, b12, w22, b22,
                 wc1, bc1, wc2, bc2, wc3, bc3,
                 out_ref, ohs_ref):
    h = x_ref[...]
    h = _edge_conv(h, wd0[...], wb0[...], b10[...], w20[...], b20[...], ohs_ref)
    h = _edge_conv(h, wd1[...], wb1[...], b11[...], w21[...], b21[...], ohs_ref)
    h = _edge_conv(h, wd2[...], wb2[...], b12[...], w22[...], b22[...], ohs_ref)
    mean = jnp.mean(h, axis=1)
    mx = jnp.max(h, axis=1)
    z = jnp.concatenate([mean, mx], axis=1)
    z = jnp.maximum(jnp.dot(z, wc1[...], precision=_HI) + bc1[...], 0.0)
    z = jnp.maximum(jnp.dot(z, wc2[...], precision=_HI) + bc2[...], 0.0)
    out_ref[...] = (jnp.dot(z, wc3[...], precision=_HI) + bc3[...])[:, None]


def _fold_edge(p):
    s1 = p["g1"] / jnp.sqrt(1.0 + EPS)
    w1f = p["W1"] * s1[None, :]
    b1f = p["b1"] * s1 + p["be1"]
    c = p["W1"].shape[0] // 2
    w1a, w1b = w1f[:c], w1f[c:]
    s2 = p["g2"] / jnp.sqrt(1.0 + EPS)
    w2f = p["W2"] * s2[None, :]
    b2f = p["b2"] * s2 + p["be2"]
    return w1a - w1b, w1b, b1f[None, :], w2f, b2f[None, :]


def kernel(x, batch, params):
    xb = jnp.pad(x.reshape(B, P, 6), ((0, 0), (0, 0), (0, 2)))

    wd0, wb0, b10, w20, b20 = _fold_edge(params["conv0"])
    wd0 = jnp.pad(wd0, ((0, 2), (0, 0)))
    wb0 = jnp.pad(wb0, ((0, 2), (0, 0)))
    wd1, wb1, b11, w21, b21 = _fold_edge(params["conv1"])
    wd2, wb2, b12, w22, b22 = _fold_edge(params["conv2"])

    c = params["cls"]
    s1 = c["g1"] / jnp.sqrt(1.0 + EPS)
    wc1 = c["W1"] * s1[None, :]
    bc1 = (c["b1"] * s1 + c["be1"])[None, :]
    s2 = c["g2"] / jnp.sqrt(1.0 + EPS)
    wc2 = c["W2"] * s2[None, :]
    bc2 = (c["b2"] * s2 + c["be2"])[None, :]
    wc3 = jnp.pad(c["W3"], ((0, 0), (0, 126)))
    bc3 = jnp.pad(c["b3"], (0, 126))[None, :]

    ops = [wd0, wb0, b10, w20, b20,
           wd1, wb1, b11, w21, b21,
           wd2, wb2, b12, w22, b22,
           wc1, bc1, wc2, bc2, wc3, bc3]

    def full(a):
        return pl.BlockSpec(a.shape, lambda i: (0,) * a.ndim)

    out = pl.pallas_call(
        _pnet_kernel,
        grid=(B // G,),
        in_specs=[pl.BlockSpec((G, P, 8), lambda i: (i, 0, 0))] +
                 [full(a) for a in ops],
        out_specs=pl.BlockSpec((G, 1, 128), lambda i: (i, 0, 0)),
        out_shape=jax.ShapeDtypeStruct((B, 1, 128), jnp.float32),
        scratch_shapes=[pltpu.VMEM((G, K, P, P), jnp.float32)],
    )(xb, *ops)
    return out.reshape(B, 128)[:, :2]


# speedup vs baseline: 23.5847x; 1.2976x over previous
import jax
import jax.numpy as jnp
from jax import lax
from jax.experimental import pallas as pl
from jax.experimental.pallas import tpu as pltpu

B, P, K = 32, 128, 16
G = 8
_KCHUNKS = 2
EPS = 1e-5
_HI = lax.Precision.HIGHEST
_MED = lax.Precision.DEFAULT


def _edge_conv(h, wd, wb, b1, w2, b2, ohs_ref):
    cin = h.shape[-1]
    cout = w2.shape[1]
    sq = jnp.sum(h * h, axis=2, keepdims=True)
    hf = h.reshape(G * P, cin)
    a = (jnp.dot(hf, wd, precision=_HI) + b1).reshape(G, P, cout)
    bv = jnp.dot(hf, wb, precision=_HI).reshape(G, P, cout)

    d = jnp.concatenate(
        [(sq[g] + sq[g].T
          - 2.0 * lax.dot_general(h[g], h[g], (((1,), (1,)), ((), ())),
                                  precision=_HI))[None]
         for g in range(G)], axis=0)

    col = lax.broadcasted_iota(jnp.int32, (G, P, P), 2).astype(jnp.float32)

    dw = d
    for k in range(K):
        m = jnp.min(dw, axis=2, keepdims=True)
        eq = dw == m
        first = jnp.min(jnp.where(eq, col, jnp.float32(P)), axis=2,
                        keepdims=True)
        oh = col == first
        ohs_ref[k] = oh.astype(jnp.float32)
        dw = jnp.where(oh, jnp.float32(jnp.inf), dw)

    kc = K // _KCHUNKS
    out = None
    for c in range(_KCHUNKS):
        gk = jnp.concatenate(
            [jnp.dot(ohs_ref[pl.ds(c * kc, kc), g].reshape(kc * P, P),
                     bv[g], precision=_MED)
             for g in range(G)], axis=0)
        a_rep = jnp.broadcast_to(a[:, None], (G, kc, P, cout)).reshape(
            G * kc * P, cout)
        r1 = jnp.maximum(a_rep + gk, 0.0)
        h2 = jnp.dot(r1, w2, precision=_MED) + b2
        part = jnp.max(jnp.maximum(h2, 0.0).reshape(G, kc, P, cout), axis=1)
        out = part if out is None else jnp.maximum(out, part)
    return out


def _pnet_kernel(x_ref,
                 wd0, wb0, b10, w20, b20,
                 wd1, wb1, b11, w21, b21,
                 wd2, wb2, b12, w22, b22,
                 wc1, bc1, wc2, bc2, wc3, bc3,
                 out_ref, ohs_ref):
    h = x_ref[...]
    h = _edge_conv(h, wd0[...], wb0[...], b10[...], w20[...], b20[...], ohs_ref)
    h = _edge_conv(h, wd1[...], wb1[...], b11[...], w21[...], b21[...], ohs_ref)
    h = _edge_conv(h, wd2[...], wb2[...], b12[...], w22[...], b22[...], ohs_ref)
    mean = jnp.mean(h, axis=1)
    mx = jnp.max(h, axis=1)
    z = jnp.concatenate([mean, mx], axis=1)
    z = jnp.maximum(jnp.dot(z, wc1[...], precision=_HI) + bc1[...], 0.0)
    z = jnp.maximum(jnp.dot(z, wc2[...], precision=_HI) + bc2[...], 0.0)
    out_ref[...] = (jnp.dot(z, wc3[...], precision=_HI) + bc3[...])[:, None]


def _fold_edge(p):
    s1 = p["g1"] / jnp.sqrt(1.0 + EPS)
    w1f = p["W1"] * s1[None, :]
    b1f = p["b1"] * s1 + p["be1"]
    c = p["W1"].shape[0] // 2
    w1a, w1b = w1f[:c], w1f[c:]
    s2 = p["g2"] / jnp.sqrt(1.0 + EPS)
    w2f = p["W2"] * s2[None, :]
    b2f = p["b2"] * s2 + p["be2"]
    return w1a - w1b, w1b, b1f[None, :], w2f, b2f[None, :]


def kernel(x, batch, params):
    xb = jnp.pad(x.reshape(B, P, 6), ((0, 0), (0, 0), (0, 2)))

    wd0, wb0, b10, w20, b20 = _fold_edge(params["conv0"])
    wd0 = jnp.pad(wd0, ((0, 2), (0, 0)))
    wb0 = jnp.pad(wb0, ((0, 2), (0, 0)))
    wd1, wb1, b11, w21, b21 = _fold_edge(params["conv1"])
    wd2, wb2, b12, w22, b22 = _fold_edge(params["conv2"])

    c = params["cls"]
    s1 = c["g1"] / jnp.sqrt(1.0 + EPS)
    wc1 = c["W1"] * s1[None, :]
    bc1 = (c["b1"] * s1 + c["be1"])[None, :]
    s2 = c["g2"] / jnp.sqrt(1.0 + EPS)
    wc2 = c["W2"] * s2[None, :]
    bc2 = (c["b2"] * s2 + c["be2"])[None, :]
    wc3 = jnp.pad(c["W3"], ((0, 0), (0, 126)))
    bc3 = jnp.pad(c["b3"], (0, 126))[None, :]

    ops = [wd0, wb0, b10, w20, b20,
           wd1, wb1, b11, w21, b21,
           wd2, wb2, b12, w22, b22,
           wc1, bc1, wc2, bc2, wc3, bc3]

    def full(a):
        return pl.BlockSpec(a.shape, lambda i: (0,) * a.ndim)

    out = pl.pallas_call(
        _pnet_kernel,
        grid=(B // G,),
        in_specs=[pl.BlockSpec((G, P, 8), lambda i: (i, 0, 0))] +
                 [full(a) for a in ops],
        out_specs=pl.BlockSpec((G, 1, 128), lambda i: (i, 0, 0)),
        out_shape=jax.ShapeDtypeStruct((B, 1, 128), jnp.float32),
        scratch_shapes=[pltpu.VMEM((K, G, P, P), jnp.float32)],
    )(xb, *ops)
    return out.reshape(B, 128)[:, :2]
